# baseline (speedup 1.0000x reference)
import functools

import jax
import jax.numpy as jnp
import numpy as np
from jax.experimental import pallas as pl
from jax.experimental.pallas import tpu as pltpu

N = 4096
C = 128
SNR_LIN = 10.0
SQRT_HALF = float(np.sqrt(0.5))

KR = ((0xE14166EC, 0x9EC84F81), (0x04658493, 0x009F6A70))
KI = ((0x61F15A13, 0x246FE96D), (0x65D0DF45, 0xA542AECB))
KN = ((0xA1495F6E, 0x9D577F1C), (0x840A05C2, 0x088E666D))

BM = 256
BK = 128


def _threefry_bits(k1, k2, counts_lo):
    u32 = jnp.uint32
    ks0 = u32(k1)
    ks1 = u32(k2)
    ks2 = u32(k1 ^ k2 ^ 0x1BD11BDA)

    x0 = jnp.full_like(counts_lo, ks0)
    x1 = counts_lo + ks1

    def rotl(v, r):
        return (v << u32(r)) | jax.lax.shift_right_logical(v, u32(32 - r))

    def four_rounds(x0, x1, rots):
        for r in rots:
            x0 = x0 + x1
            x1 = rotl(x1, r)
            x1 = x0 ^ x1
        return x0, x1

    ra = (13, 15, 26, 6)
    rb = (17, 29, 16, 24)
    x0, x1 = four_rounds(x0, x1, ra)
    x0, x1 = x0 + ks1, x1 + ks2 + u32(1)
    x0, x1 = four_rounds(x0, x1, rb)
    x0, x1 = x0 + ks2, x1 + ks0 + u32(2)
    x0, x1 = four_rounds(x0, x1, ra)
    x0, x1 = x0 + ks0, x1 + ks1 + u32(3)
    x0, x1 = four_rounds(x0, x1, rb)
    x0, x1 = x0 + ks1, x1 + ks2 + u32(4)
    x0, x1 = four_rounds(x0, x1, ra)
    x0, x1 = x0 + ks2, x1 + ks0 + u32(5)
    return x0 ^ x1


_EI = (8.700420910219156e-05, -0.0024035334374042767, 0.02559820049213798,
       -0.13004125157596383, 0.2922721121303683, -0.043370015036959676,
       0.11334836292712905, 0.871138225951577)


def _erfinv_from_counts(key, counts_lo):
    bits = _threefry_bits(key[0], key[1], counts_lo)
    fbits = jax.lax.shift_right_logical(bits, jnp.uint32(9)) | jnp.uint32(0x3F800000)
    f = jax.lax.bitcast_convert_type(fbits, jnp.float32)
    u = jnp.maximum(jnp.float32(-0.9999999403953552), f * 2.0 - 3.0)
    v = jnp.sqrt(-jnp.log(1.0 - u * u))
    p = jnp.float32(_EI[0])
    for c in _EI[1:]:
        p = jnp.float32(c) + p * v
    return p * u


def _shift_matmul_kernel(adj_ref, x_ref, y_ref, acc_ref, *, kr, ki, n_k):
    i = pl.program_id(0)
    k = pl.program_id(1)
    rows = jax.lax.broadcasted_iota(jnp.uint32, (BM, BK), 0)
    cols = jax.lax.broadcasted_iota(jnp.uint32, (BM, BK), 1)
    base = (i.astype(jnp.uint32) * jnp.uint32(BM)) * jnp.uint32(N) + \
        k.astype(jnp.uint32) * jnp.uint32(BK)
    flat = base + rows * jnp.uint32(N) + cols
    re = _erfinv_from_counts(kr, flat)
    im = _erfinv_from_counts(ki, flat)
    fad = jnp.sqrt(re * re + im * im) * SQRT_HALF
    s = adj_ref[...] * fad
    contrib = jax.lax.dot_general(
        s, x_ref[...], (((1,), (0,)), ((), ())),
        preferred_element_type=jnp.float32,
        precision=jax.lax.Precision.HIGHEST)

    @pl.when(k == 0)
    def _():
        acc_ref[...] = contrib

    @pl.when(k != 0)
    def _():
        acc_ref[...] += contrib

    @pl.when(k == n_k - 1)
    def _():
        y_ref[...] = acc_ref[...]


def _shift_matmul(adj, x2d, kr, ki):
    n_i, n_k = N // BM, N // BK
    body = functools.partial(_shift_matmul_kernel, kr=kr, ki=ki, n_k=n_k)
    return pl.pallas_call(
        body,
        grid=(n_i, n_k),
        in_specs=[
            pl.BlockSpec((BM, BK), lambda i, k: (i, k)),
            pl.BlockSpec((BK, C), lambda i, k: (k, 0)),
        ],
        out_specs=pl.BlockSpec((BM, C), lambda i, k: (i, 0)),
        out_shape=jax.ShapeDtypeStruct((N, C), jnp.float32),
        scratch_shapes=[pltpu.VMEM((BM, C), jnp.float32)],
        compiler_params=pltpu.CompilerParams(
            dimension_semantics=("parallel", "arbitrary")),
    )(adj, x2d)


def _combine_kernel(y_ref, w_ref, prev_ref, x_out_ref, out_ref, *, kn):
    y = y_ref[...]
    x_power = jnp.sum(y * y) / jnp.float32(N * C)
    std = jnp.sqrt(x_power / jnp.float32(SNR_LIN))
    rows = jax.lax.broadcasted_iota(jnp.uint32, (N, C), 0)
    cols = jax.lax.broadcasted_iota(jnp.uint32, (N, C), 1)
    noise = _erfinv_from_counts(kn, rows * jnp.uint32(C) + cols)
    x_new = y + noise * (jnp.float32(1.4142135381698608) * std)
    x_out_ref[...] = x_new
    contrib = jax.lax.dot_general(
        x_new, w_ref[...], (((1,), (1,)), ((), ())),
        preferred_element_type=jnp.float32,
        precision=jax.lax.Precision.HIGHEST)
    out_ref[...] = prev_ref[...] + contrib


def _combine(y, w, prev, kn):
    body = functools.partial(_combine_kernel, kn=kn)
    return pl.pallas_call(
        body,
        in_specs=[pl.BlockSpec(memory_space=pltpu.VMEM)] * 3,
        out_specs=[pl.BlockSpec(memory_space=pltpu.VMEM)] * 2,
        out_shape=[
            jax.ShapeDtypeStruct((N, C), jnp.float32),
            jax.ShapeDtypeStruct((N, C), jnp.float32),
        ],
    )(y, w, prev)


def kernel(x, adj, W0, W1):
    x2d = x[0]
    y0 = _shift_matmul(adj, x2d, KR[0], KI[0])
    x1, out0 = _combine(y0, W0, jnp.zeros((N, C), jnp.float32), KN[0])
    y1 = _shift_matmul(adj, x1, KR[1], KI[1])
    _, out = _combine(y1, W1, out0, KN[1])
    return out[None]


# speedup vs baseline: 1.1512x; 1.1512x over previous
import functools

import jax
import jax.numpy as jnp
import numpy as np
from jax.experimental import pallas as pl
from jax.experimental.pallas import tpu as pltpu

N = 4096
C = 128
SNR_LIN = 10.0
SQRT_HALF = float(np.sqrt(0.5))

KR = ((0xE14166EC, 0x9EC84F81), (0x04658493, 0x009F6A70))
KI = ((0x61F15A13, 0x246FE96D), (0x65D0DF45, 0xA542AECB))
KN = ((0xA1495F6E, 0x9D577F1C), (0x840A05C2, 0x088E666D))

BM = 512
BK = 256


def _threefry_bits(k1, k2, counts_lo):
    u32 = jnp.uint32
    ks0 = u32(k1)
    ks1 = u32(k2)
    ks2 = u32(k1 ^ k2 ^ 0x1BD11BDA)

    x0 = jnp.full_like(counts_lo, ks0)
    x1 = counts_lo + ks1

    def rotl(v, r):
        return (v << u32(r)) | jax.lax.shift_right_logical(v, u32(32 - r))

    def four_rounds(x0, x1, rots):
        for r in rots:
            x0 = x0 + x1
            x1 = rotl(x1, r)
            x1 = x0 ^ x1
        return x0, x1

    ra = (13, 15, 26, 6)
    rb = (17, 29, 16, 24)
    x0, x1 = four_rounds(x0, x1, ra)
    x0, x1 = x0 + ks1, x1 + ks2 + u32(1)
    x0, x1 = four_rounds(x0, x1, rb)
    x0, x1 = x0 + ks2, x1 + ks0 + u32(2)
    x0, x1 = four_rounds(x0, x1, ra)
    x0, x1 = x0 + ks0, x1 + ks1 + u32(3)
    x0, x1 = four_rounds(x0, x1, rb)
    x0, x1 = x0 + ks1, x1 + ks2 + u32(4)
    x0, x1 = four_rounds(x0, x1, ra)
    x0, x1 = x0 + ks2, x1 + ks0 + u32(5)
    return x0 ^ x1


_EI = (8.700420910219156e-05, -0.0024035334374042767, 0.02559820049213798,
       -0.13004125157596383, 0.2922721121303683, -0.043370015036959676,
       0.11334836292712905, 0.871138225951577)


def _erfinv_from_counts(key, counts_lo):
    bits = _threefry_bits(key[0], key[1], counts_lo)
    fbits = jax.lax.shift_right_logical(bits, jnp.uint32(9)) | jnp.uint32(0x3F800000)
    f = jax.lax.bitcast_convert_type(fbits, jnp.float32)
    u = jnp.maximum(jnp.float32(-0.9999999403953552), f * 2.0 - 3.0)
    v = jnp.sqrt(-jnp.log(1.0 - u * u))
    p = jnp.float32(_EI[0])
    for c in _EI[1:]:
        p = jnp.float32(c) + p * v
    return p * u


def _shift_matmul_kernel(adj_ref, x_ref, y_ref, acc_ref, *, kr, ki, n_k):
    i = pl.program_id(0)
    k = pl.program_id(1)
    rows = jax.lax.broadcasted_iota(jnp.uint32, (BM, BK), 0)
    cols = jax.lax.broadcasted_iota(jnp.uint32, (BM, BK), 1)
    base = (i.astype(jnp.uint32) * jnp.uint32(BM)) * jnp.uint32(N) + \
        k.astype(jnp.uint32) * jnp.uint32(BK)
    flat = base + rows * jnp.uint32(N) + cols
    re = _erfinv_from_counts(kr, flat)
    im = _erfinv_from_counts(ki, flat)
    fad = jnp.sqrt(re * re + im * im) * SQRT_HALF
    s = adj_ref[...] * fad
    contrib = jax.lax.dot_general(
        s, x_ref[...], (((1,), (0,)), ((), ())),
        preferred_element_type=jnp.float32,
        precision=jax.lax.Precision.HIGHEST)

    @pl.when(k == 0)
    def _():
        acc_ref[...] = contrib

    @pl.when(k != 0)
    def _():
        acc_ref[...] += contrib

    @pl.when(k == n_k - 1)
    def _():
        y_ref[...] = acc_ref[...]


def _shift_matmul(adj, x2d, kr, ki):
    n_i, n_k = N // BM, N // BK
    body = functools.partial(_shift_matmul_kernel, kr=kr, ki=ki, n_k=n_k)
    return pl.pallas_call(
        body,
        grid=(n_i, n_k),
        in_specs=[
            pl.BlockSpec((BM, BK), lambda i, k: (i, k)),
            pl.BlockSpec((BK, C), lambda i, k: (k, 0)),
        ],
        out_specs=pl.BlockSpec((BM, C), lambda i, k: (i, 0)),
        out_shape=jax.ShapeDtypeStruct((N, C), jnp.float32),
        scratch_shapes=[pltpu.VMEM((BM, C), jnp.float32)],
        compiler_params=pltpu.CompilerParams(
            dimension_semantics=("parallel", "arbitrary")),
    )(adj, x2d)


def _combine_kernel(y_ref, w_ref, prev_ref, x_out_ref, out_ref, *, kn):
    y = y_ref[...]
    x_power = jnp.sum(y * y) / jnp.float32(N * C)
    std = jnp.sqrt(x_power / jnp.float32(SNR_LIN))
    rows = jax.lax.broadcasted_iota(jnp.uint32, (N, C), 0)
    cols = jax.lax.broadcasted_iota(jnp.uint32, (N, C), 1)
    noise = _erfinv_from_counts(kn, rows * jnp.uint32(C) + cols)
    x_new = y + noise * (jnp.float32(1.4142135381698608) * std)
    x_out_ref[...] = x_new
    contrib = jax.lax.dot_general(
        x_new, w_ref[...], (((1,), (1,)), ((), ())),
        preferred_element_type=jnp.float32,
        precision=jax.lax.Precision.HIGHEST)
    out_ref[...] = prev_ref[...] + contrib


def _combine(y, w, prev, kn):
    body = functools.partial(_combine_kernel, kn=kn)
    return pl.pallas_call(
        body,
        in_specs=[pl.BlockSpec(memory_space=pltpu.VMEM)] * 3,
        out_specs=[pl.BlockSpec(memory_space=pltpu.VMEM)] * 2,
        out_shape=[
            jax.ShapeDtypeStruct((N, C), jnp.float32),
            jax.ShapeDtypeStruct((N, C), jnp.float32),
        ],
    )(y, w, prev)


def kernel(x, adj, W0, W1):
    x2d = x[0]
    y0 = _shift_matmul(adj, x2d, KR[0], KI[0])
    x1, out0 = _combine(y0, W0, jnp.zeros((N, C), jnp.float32), KN[0])
    y1 = _shift_matmul(adj, x1, KR[1], KI[1])
    _, out = _combine(y1, W1, out0, KN[1])
    return out[None]
